# Initial kernel scaffold; baseline (speedup 1.0000x reference)
import jax
import jax.numpy as jnp
from jax.experimental import pallas as pl


def kernel(input_, weight):
    raise NotImplementedError("write your pallas kernel here")


# speedup vs baseline: 1.8429x; 1.8429x over previous
import functools

import jax
import jax.numpy as jnp
from jax import lax
from jax.experimental import pallas as pl
from jax.experimental.pallas import tpu as pltpu
from jax.experimental.pallas import tpu_sc as plsc

V = 1000000
D = 64
B = 16384 * 50
NW = 32
PER_W = B // NW
SUB = 128
K = 8
C = SUB * K
STEPS = PER_W // C

_mesh = plsc.VectorSubcoreMesh(core_axis_name="c", subcore_axis_name="s")


@functools.partial(
    pl.kernel,
    out_type=jax.ShapeDtypeStruct((B, D), jnp.float32),
    mesh=_mesh,
    scratch_types=[
        pltpu.VMEM((K, SUB), jnp.int32),
        pltpu.VMEM((C, D), jnp.float32),
        pltpu.SemaphoreType.DMA,
    ],
    compiler_params=pltpu.CompilerParams(use_tc_tiling_on_sc=False),
)
def _emb_gather(idx_hbm, table_hbm, out_hbm, idx_v, rows_v, sem):
    wid = lax.axis_index("s") * 2 + lax.axis_index("c")
    base = wid * PER_W

    def step(i, carry):
        off = pl.multiple_of(base + i * C, C)
        pltpu.sync_copy(idx_hbm.at[pl.ds(pl.multiple_of(off // SUB, K), K)], idx_v)
        handles = [
            pltpu.async_copy(
                table_hbm.at[idx_v.at[j]],
                rows_v.at[pl.ds(j * SUB, SUB)],
                sem,
            )
            for j in range(K)
        ]
        for h in handles:
            h.wait()
        pltpu.sync_copy(rows_v, out_hbm.at[pl.ds(off, C)])
        return carry

    lax.fori_loop(0, STEPS, step, 0)


def kernel(input_, weight):
    idx = jnp.reshape(input_.astype(jnp.int32), (B // SUB, SUB))
    out = _emb_gather(idx, weight)
    return jnp.reshape(out, (*input_.shape, D))


# speedup vs baseline: 1.8474x; 1.0025x over previous
import functools

import jax
import jax.numpy as jnp
from jax import lax
from jax.experimental import pallas as pl
from jax.experimental.pallas import tpu as pltpu
from jax.experimental.pallas import tpu_sc as plsc

V = 1000000
D = 64
B = 16384 * 50
NW = 32
PER_W = B // NW
SUB = 128
KC = 4
C = SUB * KC
STEPS = PER_W // C

_mesh = plsc.VectorSubcoreMesh(core_axis_name="c", subcore_axis_name="s")


@functools.partial(
    pl.kernel,
    out_type=jax.ShapeDtypeStruct((B, D), jnp.float32),
    mesh=_mesh,
    scratch_types=[
        pltpu.VMEM((2, C), jnp.int32),
        pltpu.VMEM((2, C, D), jnp.float32),
        pltpu.SemaphoreType.DMA((2,)),
        pltpu.SemaphoreType.DMA,
    ],
    compiler_params=pltpu.CompilerParams(use_tc_tiling_on_sc=False),
)
def _emb_gather(idx_hbm, table_hbm, out_hbm, idx_v, rows_v, sem_g, sem_w):
    wid = lax.axis_index("s") * 2 + lax.axis_index("c")
    base = wid * PER_W

    def chunk_off(i):
        return pl.multiple_of(base + i * C, C)

    def fire_gathers(s, i):
        off = chunk_off(i)
        pltpu.sync_copy(idx_hbm.at[pl.ds(off, C)], idx_v.at[s])
        for j in range(KC):
            pltpu.async_copy(
                table_hbm.at[idx_v.at[s, pl.ds(j * SUB, SUB)]],
                rows_v.at[s, pl.ds(j * SUB, SUB)],
                sem_g.at[s],
            )

    def wait_gathers(s):
        for j in range(KC):
            pltpu.make_async_copy(
                table_hbm.at[idx_v.at[s, pl.ds(j * SUB, SUB)]],
                rows_v.at[s, pl.ds(j * SUB, SUB)],
                sem_g.at[s],
            ).wait()

    def fire_wb(s, i):
        pltpu.async_copy(rows_v.at[s], out_hbm.at[pl.ds(chunk_off(i), C)], sem_w)

    def wait_wb(s, i):
        pltpu.make_async_copy(
            rows_v.at[s], out_hbm.at[pl.ds(chunk_off(i), C)], sem_w
        ).wait()

    def step(i, carry):
        s = lax.rem(i, 2)
        ps = lax.rem(i + 1, 2)

        @pl.when(i >= 1)
        def _():
            wait_gathers(ps)
            fire_wb(ps, i - 1)

        @pl.when(i >= 2)
        def _():
            wait_wb(s, i - 2)

        fire_gathers(s, i)
        return carry

    lax.fori_loop(0, STEPS, step, 0)

    ls = (STEPS - 1) % 2
    wait_gathers(ls)
    fire_wb(ls, STEPS - 1)
    wait_wb(1 - ls, STEPS - 2)
    wait_wb(ls, STEPS - 1)


def kernel(input_, weight):
    idx = jnp.reshape(input_.astype(jnp.int32), (B,))
    out = _emb_gather(idx, weight)
    return jnp.reshape(out, (*input_.shape, D))


# speedup vs baseline: 1.8570x; 1.0051x over previous
import functools

import jax
import jax.numpy as jnp
from jax import lax
from jax.experimental import pallas as pl
from jax.experimental.pallas import tpu as pltpu
from jax.experimental.pallas import tpu_sc as plsc

V = 1000000
D = 64
S = 16384
W = 50
NW = 32
PER_W = S // NW
CS = 16
STEPS = PER_W // CS

_mesh = plsc.VectorSubcoreMesh(core_axis_name="c", subcore_axis_name="s")


@functools.partial(
    pl.kernel,
    out_type=jax.ShapeDtypeStruct((S, W, D), jnp.float32),
    mesh=_mesh,
    scratch_types=[
        pltpu.VMEM((2, CS, W), jnp.int32),
        pltpu.VMEM((2, CS, W, D), jnp.float32),
        pltpu.SemaphoreType.DMA((2,)),
        pltpu.SemaphoreType.DMA,
    ],
    compiler_params=pltpu.CompilerParams(use_tc_tiling_on_sc=False),
)
def _emb_gather(idx_hbm, table_hbm, out_hbm, idx_v, rows_v, sem_g, sem_w):
    wid = lax.axis_index("s") * 2 + lax.axis_index("c")
    base = wid * PER_W

    def chunk_off(i):
        return pl.multiple_of(base + i * CS, CS)

    def fire_gathers(s, i):
        off = chunk_off(i)
        pltpu.sync_copy(idx_hbm.at[pl.ds(off, CS)], idx_v.at[s])
        for k in range(CS):
            pltpu.async_copy(
                table_hbm.at[idx_v.at[s, k]],
                rows_v.at[s, k],
                sem_g.at[s],
            )

    def wait_gathers(s):
        for k in range(CS):
            pltpu.make_async_copy(
                table_hbm.at[idx_v.at[s, k]],
                rows_v.at[s, k],
                sem_g.at[s],
            ).wait()

    def fire_wb(s, i):
        pltpu.async_copy(rows_v.at[s], out_hbm.at[pl.ds(chunk_off(i), CS)], sem_w)

    def wait_wb(s, i):
        pltpu.make_async_copy(
            rows_v.at[s], out_hbm.at[pl.ds(chunk_off(i), CS)], sem_w
        ).wait()

    def step(i, carry):
        s = lax.rem(i, 2)
        ps = lax.rem(i + 1, 2)

        @pl.when(i >= 1)
        def _():
            wait_gathers(ps)
            fire_wb(ps, i - 1)

        @pl.when(i >= 2)
        def _():
            wait_wb(s, i - 2)

        fire_gathers(s, i)
        return carry

    lax.fori_loop(0, STEPS, step, 0)

    ls = (STEPS - 1) % 2
    wait_gathers(ls)
    fire_wb(ls, STEPS - 1)
    wait_wb(1 - ls, STEPS - 2)
    wait_wb(ls, STEPS - 1)


def kernel(input_, weight):
    return _emb_gather(input_.astype(jnp.int32), weight)
